# baseline (speedup 1.0000x reference)
import functools

import jax
import jax.numpy as jnp
from jax import lax
from jax.experimental import pallas as pl
from jax.experimental.pallas import tpu as pltpu
from jax.experimental.pallas import tpu_sc as plsc

B = 16384
D = 16
NS = 16
CH = 128
KC = B // (NS * CH)
BPW = KC * CH
LANES = 16


def _sc_body(uidx_hbm, midx_hbm, uemb_hbm, memb_hbm, ubias_hbm, mbias_hbm,
             gb_hbm, out_hbm,
             uidx_v, midx_v, urows_v, mrows_v, ubias_v, mbias_v, out_v,
             acc_v, partials_v, gb_v, shared, sem):
    sid = lax.axis_index("s")

    pltpu.sync_copy(uidx_hbm.at[sid], uidx_v)
    pltpu.sync_copy(midx_hbm.at[sid], midx_v)
    pltpu.sync_copy(gb_hbm, gb_v)

    copies = []
    for j in range(KC):
        copies.append(pltpu.async_copy(uemb_hbm.at[uidx_v.at[j]], urows_v.at[j], sem))
        copies.append(pltpu.async_copy(memb_hbm.at[midx_v.at[j]], mrows_v.at[j], sem))
        copies.append(pltpu.async_copy(ubias_hbm.at[uidx_v.at[j]], ubias_v.at[j], sem))
        copies.append(pltpu.async_copy(mbias_hbm.at[midx_v.at[j]], mbias_v.at[j], sem))
    for c in copies:
        c.wait()

    acc = jnp.zeros((LANES,), jnp.float32)
    for j in range(KC):
        def step(i, a, j=j):
            return a + urows_v[j, i, :] * mrows_v[j, i, :]
        acc = lax.fori_loop(0, CH, step, acc)
    acc_v[...] = acc

    pltpu.sync_copy(acc_v, shared.at[sid])
    plsc.subcore_barrier()
    pltpu.sync_copy(shared, partials_v)
    tot = jnp.zeros((LANES,), jnp.float32)
    for t in range(NS):
        tot = tot + partials_v[t, :]
    lane = lax.iota(jnp.int32, LANES)
    for sh in (1, 2, 4, 8):
        acc_v[...] = tot
        tot = tot + plsc.load_gather(acc_v, [lane ^ sh])

    base = tot + gb_v[...]
    for j in range(KC):
        for t in range(CH // LANES):
            sl = pl.ds(t * LANES, LANES)
            out_v[j, sl] = ubias_v[j, sl] + mbias_v[j, sl] + base
    pltpu.sync_copy(out_v, out_hbm.at[sid])


@jax.jit
def _run(uidx, midx, uemb, memb, ubias, mbias, gb16):
    mesh = plsc.VectorSubcoreMesh(core_axis_name="c", subcore_axis_name="s",
                                  num_cores=1)
    f = pl.kernel(
        _sc_body,
        out_type=jax.ShapeDtypeStruct((NS, KC, CH), jnp.float32),
        mesh=mesh,
        scratch_types=[
            pltpu.VMEM((KC, CH), jnp.int32),
            pltpu.VMEM((KC, CH), jnp.int32),
            pltpu.VMEM((KC, CH, D), jnp.float32),
            pltpu.VMEM((KC, CH, D), jnp.float32),
            pltpu.VMEM((KC, CH), jnp.float32),
            pltpu.VMEM((KC, CH), jnp.float32),
            pltpu.VMEM((KC, CH), jnp.float32),
            pltpu.VMEM((LANES,), jnp.float32),
            pltpu.VMEM((NS, LANES), jnp.float32),
            pltpu.VMEM((LANES,), jnp.float32),
            pltpu.VMEM_SHARED((NS, LANES), jnp.float32),
            pltpu.SemaphoreType.DMA,
        ],
        compiler_params=pltpu.CompilerParams(needs_layout_passes=False,
                                             use_tc_tiling_on_sc=False),
    )
    return f(uidx, midx, uemb, memb, ubias, mbias, gb16)


def kernel(inputs, user_embedding, movie_embedding, user_bias_table,
           movie_bias_table, global_bias):
    idx = inputs.astype(jnp.int32)
    uidx = idx[:, 0].reshape(NS, KC, CH)
    midx = idx[:, 1].reshape(NS, KC, CH)
    ubias = user_bias_table.reshape(-1)
    mbias = movie_bias_table.reshape(-1)
    gb16 = jnp.broadcast_to(global_bias.astype(jnp.float32), (LANES,))
    out = _run(uidx, midx, user_embedding, movie_embedding, ubias, mbias, gb16)
    return out.reshape(B)
